# baseline (speedup 1.0000x reference)
import jax
import jax.numpy as jnp
from jax import lax
from jax.experimental import pallas as pl
from jax.experimental.pallas import tpu as pltpu


_DOT_TRANS_B = (((1,), (1,)), ((), ()))


def _fused_noisy_kernel(x_ref, mu_w_ref, sig_w_ref, eps_oc_ref, eps_in_ref,
                        mu_b_ref, sig_b_ref, eps_or_ref, o_ref):
    k = pl.program_id(0)
    scale = eps_oc_ref[...] * eps_in_ref[...]
    w_eff = mu_w_ref[...] + sig_w_ref[...] * scale
    y = lax.dot_general(x_ref[...], w_eff, _DOT_TRANS_B,
                        preferred_element_type=jnp.float32)

    @pl.when(k == 0)
    def _first():
        b_eff = sig_b_ref[...] * eps_or_ref[...] + mu_b_ref[...]
        o_ref[...] = y + b_eff

    @pl.when(k != 0)
    def _rest():
        o_ref[...] += y


def kernel(x, mu_weight, sigma_weight, mu_bias, sigma_bias, eps_in, eps_out):
    B, F_in = x.shape
    F_out = mu_bias.shape[0]

    x_f = x.astype(jnp.float32)
    mu_w = mu_weight.astype(jnp.float32)
    sig_w = sigma_weight.astype(jnp.float32)
    eps_in_row = eps_in.reshape(1, F_in).astype(jnp.float32)
    eps_out_col = eps_out.reshape(F_out, 1).astype(jnp.float32)
    eps_out_row = eps_out.reshape(1, F_out).astype(jnp.float32)
    mu_b_row = mu_bias.reshape(1, F_out).astype(jnp.float32)
    sig_b_row = sigma_bias.reshape(1, F_out).astype(jnp.float32)

    bk = 256 if F_in % 256 == 0 else F_in
    grid = (F_in // bk,)

    return pl.pallas_call(
        _fused_noisy_kernel,
        out_shape=jax.ShapeDtypeStruct((B, F_out), jnp.float32),
        grid=grid,
        in_specs=[
            pl.BlockSpec((B, bk), lambda k: (0, k)),
            pl.BlockSpec((F_out, bk), lambda k: (0, k)),
            pl.BlockSpec((F_out, bk), lambda k: (0, k)),
            pl.BlockSpec((F_out, 1), lambda k: (0, 0)),
            pl.BlockSpec((1, bk), lambda k: (0, k)),
            pl.BlockSpec((1, F_out), lambda k: (0, 0)),
            pl.BlockSpec((1, F_out), lambda k: (0, 0)),
            pl.BlockSpec((1, F_out), lambda k: (0, 0)),
        ],
        out_specs=pl.BlockSpec((B, F_out), lambda k: (0, 0)),
        compiler_params=pltpu.CompilerParams(
            dimension_semantics=("arbitrary",),
            vmem_limit_bytes=64 * 1024 * 1024,
        ),
    )(x_f, mu_w, sig_w, eps_out_col, eps_in_row, mu_b_row, sig_b_row,
      eps_out_row)


# speedup vs baseline: 1.3469x; 1.3469x over previous
import jax
import jax.numpy as jnp
from jax import lax
from jax.experimental import pallas as pl
from jax.experimental.pallas import tpu as pltpu


_DOT_TRANS_B = (((1,), (1,)), ((), ()))


def _fused_noisy_kernel(x_ref, mu_w_ref, sig_w_ref, eps_oc_ref, eps_in_ref,
                        mu_b_ref, sig_b_ref, eps_or_ref, o_ref):
    scale = eps_oc_ref[...] * eps_in_ref[...]
    w_eff = mu_w_ref[...] + sig_w_ref[...] * scale
    y = lax.dot_general(x_ref[...], w_eff, _DOT_TRANS_B,
                        preferred_element_type=jnp.float32)
    b_eff = sig_b_ref[...] * eps_or_ref[...] + mu_b_ref[...]
    o_ref[...] = (y + b_eff).astype(o_ref.dtype)


def kernel(x, mu_weight, sigma_weight, mu_bias, sigma_bias, eps_in, eps_out):
    B, F_in = x.shape
    F_out = mu_bias.shape[0]

    x_f = x.astype(jnp.float32)
    mu_w = mu_weight.astype(jnp.float32)
    sig_w = sigma_weight.astype(jnp.float32)
    eps_in_row = eps_in.reshape(1, F_in).astype(jnp.float32)
    eps_out_col = eps_out.reshape(F_out, 1).astype(jnp.float32)
    eps_out_row = eps_out.reshape(1, F_out).astype(jnp.float32)
    mu_b_row = mu_bias.reshape(1, F_out).astype(jnp.float32)
    sig_b_row = sigma_bias.reshape(1, F_out).astype(jnp.float32)

    tn = 256 if F_out % 256 == 0 else F_out
    grid = (F_out // tn,)

    return pl.pallas_call(
        _fused_noisy_kernel,
        out_shape=jax.ShapeDtypeStruct((B, F_out), jnp.float32),
        grid=grid,
        in_specs=[
            pl.BlockSpec((B, F_in), lambda j: (0, 0)),
            pl.BlockSpec((tn, F_in), lambda j: (j, 0)),
            pl.BlockSpec((tn, F_in), lambda j: (j, 0)),
            pl.BlockSpec((tn, 1), lambda j: (j, 0)),
            pl.BlockSpec((1, F_in), lambda j: (0, 0)),
            pl.BlockSpec((1, tn), lambda j: (0, j)),
            pl.BlockSpec((1, tn), lambda j: (0, j)),
            pl.BlockSpec((1, tn), lambda j: (0, j)),
        ],
        out_specs=pl.BlockSpec((B, tn), lambda j: (0, j)),
        compiler_params=pltpu.CompilerParams(
            dimension_semantics=("parallel",),
            vmem_limit_bytes=64 * 1024 * 1024,
        ),
    )(x_f, mu_w, sig_w, eps_out_col, eps_in_row, mu_b_row, sig_b_row,
      eps_out_row)
